# baseline (speedup 1.0000x reference)
import jax
import jax.numpy as jnp
from jax.experimental import pallas as pl
from jax.experimental.pallas import tpu as pltpu

_BATCH = 16384
_CLASSES = 1000
_KEEP = int(_BATCH * 0.7)
_COLS = 1024


def _ce_t_kernel(predt_ref, tgt_ref, out_ref):
    x = predt_ref[...]
    tgt = tgt_ref[...]
    m = jnp.max(x, axis=0, keepdims=True)
    s = jnp.sum(jnp.exp(x - m), axis=0, keepdims=True)
    lse = m + jnp.log(s)
    row = jax.lax.broadcasted_iota(jnp.int32, x.shape, 0)
    safe = jnp.clip(tgt, 0, _CLASSES - 1)
    tsel = jnp.sum(jnp.where(row == safe, x, 0.0), axis=0, keepdims=True)
    ce = lse - tsel
    ce = jnp.where(tgt == -1, 0.0, ce)
    out_ref[...] = ce


def _topk_kernel(ce_ref, out_ref):
    ce = ce_ref[...]

    def body(_, lohi):
        lo, hi = lohi
        w = hi - lo
        m1 = lo + w // 4
        m2 = lo + w // 2
        m3 = m2 + w // 4
        c1 = jnp.sum((ce >= jax.lax.bitcast_convert_type(m1, jnp.float32))
                     .astype(jnp.int32))
        c2 = jnp.sum((ce >= jax.lax.bitcast_convert_type(m2, jnp.float32))
                     .astype(jnp.int32))
        c3 = jnp.sum((ce >= jax.lax.bitcast_convert_type(m3, jnp.float32))
                     .astype(jnp.int32))
        ge1 = c1 >= _KEEP
        ge2 = c2 >= _KEEP
        ge3 = c3 >= _KEEP
        lo2 = jnp.where(ge3, m3, jnp.where(ge2, m2, jnp.where(ge1, m1, lo)))
        hi2 = jnp.where(jnp.logical_not(ge1), m1,
                        jnp.where(jnp.logical_not(ge2), m2,
                                  jnp.where(jnp.logical_not(ge3), m3, hi)))
        return lo2, hi2

    lo, _ = jax.lax.fori_loop(
        0, 18, body, (jnp.int32(0), jnp.int32(0x7F800000))
    )
    t = jax.lax.bitcast_convert_type(lo, jnp.float32)
    gt = ce > t
    cnt_gt = jnp.sum(gt.astype(jnp.int32))
    sum_gt = jnp.sum(jnp.where(gt, ce, 0.0))
    total = sum_gt + (_KEEP - cnt_gt).astype(jnp.float32) * t
    out_ref[...] = jnp.broadcast_to(total / jnp.float32(_KEEP), (1, 1))


@jax.jit
def kernel(pred, target):
    predt = pred.T
    tgt = target.astype(jnp.int32).reshape(1, _BATCH)
    grid = _BATCH // _COLS
    ce = pl.pallas_call(
        _ce_t_kernel,
        grid=(grid,),
        in_specs=[
            pl.BlockSpec((_CLASSES, _COLS), lambda i: (0, i)),
            pl.BlockSpec((1, _COLS), lambda i: (0, i)),
        ],
        out_specs=pl.BlockSpec((1, _COLS), lambda i: (0, i)),
        out_shape=jax.ShapeDtypeStruct((1, _BATCH), jnp.float32),
        compiler_params=pltpu.CompilerParams(
            dimension_semantics=("arbitrary",),
        ),
    )(predt, tgt)

    out = pl.pallas_call(
        _topk_kernel,
        out_shape=jax.ShapeDtypeStruct((1, 1), jnp.float32),
    )(ce)
    return out[0, 0]
